# baseline (speedup 1.0000x reference)
import functools
import jax
import jax.numpy as jnp
from jax import lax
from jax.experimental import pallas as pl
from jax.experimental.pallas import tpu as pltpu
from jax.experimental.pallas import tpu_sc as plsc

_B, _P, _D = 64, 1024, 768
_NW = 32
_ROWS_W = _P // _NW
_CHUNK = _ROWS_W * _D
_BATCH_STRIDE = _P * _D

_mesh = plsc.VectorSubcoreMesh(core_axis_name="c", subcore_axis_name="s")


@functools.partial(
    pl.kernel,
    mesh=_mesh,
    out_type=jax.ShapeDtypeStruct((_B * _P * _D,), jnp.float32),
    scratch_types=[
        pltpu.VMEM((_CHUNK,), jnp.float32),
        pltpu.VMEM((_CHUNK,), jnp.float32),
    ],
)
def _sc_add(x_hbm, pos_hbm, out_hbm, x_v, pos_v):
    wid = lax.axis_index("s") * 2 + lax.axis_index("c")
    pltpu.sync_copy(pos_hbm.at[pl.ds(wid * _CHUNK, _CHUNK)], pos_v)

    def per_batch(b, carry):
        off = b * _BATCH_STRIDE + wid * _CHUNK
        pltpu.sync_copy(x_hbm.at[pl.ds(off, _CHUNK)], x_v)

        def inner(i, c):
            base = i * 64
            for u in range(4):
                sl = pl.ds(base + u * 16, 16)
                x_v[sl] = x_v[sl] + pos_v[sl]
            return c

        lax.fori_loop(0, _CHUNK // 64, inner, 0)
        pltpu.sync_copy(x_v, out_hbm.at[pl.ds(off, _CHUNK)])
        return carry

    lax.fori_loop(0, _B, per_batch, 0)


def kernel(encoded_patches, pos_table):
    out = _sc_add(encoded_patches.reshape(-1), pos_table.reshape(-1))
    return out.reshape(_B, _P, _D)


# speedup vs baseline: 5.6453x; 5.6453x over previous
import jax
import jax.numpy as jnp
from jax.experimental import pallas as pl

_B, _P, _D = 64, 1024, 768
_BPB = 4


def _tc_body(x_ref, p_ref, o_ref):
    o_ref[...] = x_ref[...] + p_ref[...]


def kernel(encoded_patches, pos_table):
    return pl.pallas_call(
        _tc_body,
        grid=(_B // _BPB,),
        in_specs=[
            pl.BlockSpec((_BPB, _P, _D), lambda i: (i, 0, 0)),
            pl.BlockSpec((_P, _D), lambda i: (0, 0)),
        ],
        out_specs=pl.BlockSpec((_BPB, _P, _D), lambda i: (i, 0, 0)),
        out_shape=jax.ShapeDtypeStruct((_B, _P, _D), jnp.float32),
    )(encoded_patches, pos_table)


# speedup vs baseline: 5.6528x; 1.0013x over previous
import jax
import jax.numpy as jnp
from jax.experimental import pallas as pl

_B, _P, _D = 64, 1024, 768
_BPB = 4


def _tc_body(x_ref, p_ref, o_ref):
    o_ref[...] = x_ref[...] + p_ref[...]


def kernel(encoded_patches, pos_table):
    return pl.pallas_call(
        _tc_body,
        grid=(pl.cdiv(_B, _BPB),),
        in_specs=[
            pl.BlockSpec((_BPB, _P, _D), lambda i: (i, 0, 0)),
            pl.BlockSpec((_P, _D), lambda i: (0, 0)),
        ],
        out_specs=pl.BlockSpec((_BPB, _P, _D), lambda i: (i, 0, 0)),
        out_shape=jax.ShapeDtypeStruct((_B, _P, _D), jnp.float32),
    )(encoded_patches, pos_table)
